# Initial kernel scaffold; baseline (speedup 1.0000x reference)
import jax
import jax.numpy as jnp
from jax.experimental import pallas as pl


def kernel(x, adj, W0_0, a0_0, W0_1, a0_1, W_out, a_out):
    raise NotImplementedError("write your pallas kernel here")


# speedup vs baseline: 1.4424x; 1.4424x over previous
import functools

import jax
import jax.numpy as jnp
from jax.experimental import pallas as pl
from jax.experimental.pallas import tpu as pltpu

N = 8192
NFEAT = 256
NHID = 64
ALPHA = 0.2


def _elu(x):
    return jnp.where(x > 0, x, jnp.exp(x) - 1.0)


def _proj_kernel(x_ref, w_ref, b_ref, amat_ref, h_ref, sc_ref, mx_ref):
    i = pl.program_id(0)
    h = jnp.dot(x_ref[...], w_ref[...], preferred_element_type=jnp.float32)
    h = h + b_ref[...]
    h_ref[...] = h.astype(jnp.bfloat16)
    sc = jnp.dot(h, amat_ref[...], preferred_element_type=jnp.float32)
    sc_ref[...] = sc
    cm = jnp.max(sc, axis=0, keepdims=True)

    @pl.when(i == 0)
    def _():
        mx_ref[...] = cm

    @pl.when(i > 0)
    def _():
        mx_ref[...] = jnp.maximum(mx_ref[...], cm)


def _project(x, w_aug, bias, amat, block_rows=1024):
    n, k = x.shape
    f = w_aug.shape[1]
    grid = (n // block_rows,)
    return pl.pallas_call(
        _proj_kernel,
        grid=grid,
        in_specs=[
            pl.BlockSpec((block_rows, k), lambda i: (i, 0)),
            pl.BlockSpec((k, f), lambda i: (0, 0)),
            pl.BlockSpec((1, f), lambda i: (0, 0)),
            pl.BlockSpec((f, 128), lambda i: (0, 0)),
        ],
        out_specs=[
            pl.BlockSpec((block_rows, f), lambda i: (i, 0)),
            pl.BlockSpec((block_rows, 128), lambda i: (i, 0)),
            pl.BlockSpec((1, 128), lambda i: (0, 0)),
        ],
        out_shape=[
            jax.ShapeDtypeStruct((n, f), jnp.bfloat16),
            jax.ShapeDtypeStruct((n, 128), jnp.float32),
            jax.ShapeDtypeStruct((1, 128), jnp.float32),
        ],
        compiler_params=pltpu.CompilerParams(
            dimension_semantics=("arbitrary",),
        ),
    )(x, w_aug, bias, amat)


def _attn_kernel(adj_ref, wh_ref, sc_ref, mx_ref, ert_ref, out_ref, acc_ref,
                 *, nheads, nj, bc, final):
    j = pl.program_id(1)

    @pl.when(j == 0)
    def _():
        acc_ref[...] = jnp.zeros_like(acc_ref)

    adj = adj_ref[...]
    for h in range(nheads):
        el = sc_ref[:, 2 * h:2 * h + 1]
        erm = mx_ref[0:1, 2 * h + 1:2 * h + 2]
        t = el + erm
        m = jnp.maximum(t, ALPHA * t)
        a = jnp.exp(t - m)
        c = jnp.exp(ALPHA * t - m)
        d0 = ert_ref[h:h + 1, :] - erm
        b = jnp.exp(d0)
        d = jnp.exp(ALPHA * d0)
        p = adj * jnp.maximum(a * b, c * d)
        wh = wh_ref[pl.ds(j * bc, bc), 128 * h:128 * (h + 1)]
        acc_ref[:, 128 * h:128 * (h + 1)] += jnp.dot(
            p.astype(jnp.bfloat16), wh, preferred_element_type=jnp.float32)

    @pl.when(j == nj - 1)
    def _():
        for h in range(nheads):
            acc = acc_ref[:, 128 * h:128 * (h + 1)]
            s = jnp.maximum(acc[:, NHID:NHID + 1], 1e-30)
            hp = acc[:, :NHID] * (1.0 / s)
            if final:
                o = _elu(hp)
                mx = jnp.max(o, axis=1, keepdims=True)
                lse = jnp.log(jnp.sum(jnp.exp(o - mx), axis=1, keepdims=True))
                out_ref[...] = o - mx - lse
            else:
                out_ref[:, NHID * h:NHID * (h + 1)] = _elu(hp)


def _attention(adj, wh_aug, scores, colmax, er_t, nheads, final,
               br=256, bc=1024):
    n = adj.shape[0]
    ni, nj = n // br, n // bc
    out_cols = NHID if final else NHID * nheads
    kern = functools.partial(_attn_kernel, nheads=nheads, nj=nj, bc=bc,
                             final=final)
    return pl.pallas_call(
        kern,
        grid=(ni, nj),
        in_specs=[
            pl.BlockSpec((br, bc), lambda i, j: (i, j)),
            pl.BlockSpec((n, 128 * nheads), lambda i, j: (0, 0)),
            pl.BlockSpec((br, 128), lambda i, j: (i, 0)),
            pl.BlockSpec((1, 128), lambda i, j: (0, 0)),
            pl.BlockSpec((8, bc), lambda i, j: (0, j)),
        ],
        out_specs=pl.BlockSpec((br, out_cols), lambda i, j: (i, 0)),
        out_shape=jax.ShapeDtypeStruct((n, out_cols), jnp.float32),
        scratch_shapes=[pltpu.VMEM((br, 128 * nheads), jnp.float32)],
        compiler_params=pltpu.CompilerParams(
            dimension_semantics=("parallel", "arbitrary"),
        ),
    )(adj, wh_aug, scores, colmax, er_t)


def kernel(x, adj, W0_0, a0_0, W0_1, a0_1, W_out, a_out):
    f32 = jnp.float32

    w1 = jnp.zeros((NFEAT, 256), f32)
    w1 = w1.at[:, 0:NHID].set(W0_0).at[:, 128:128 + NHID].set(W0_1)
    b1 = jnp.zeros((1, 256), f32).at[0, NHID].set(1.0).at[0, 128 + NHID].set(1.0)
    amat1 = jnp.zeros((256, 128), f32)
    amat1 = (amat1.at[0:NHID, 0].set(a0_0[:NHID, 0])
                  .at[0:NHID, 1].set(a0_0[NHID:, 0])
                  .at[128:128 + NHID, 2].set(a0_1[:NHID, 0])
                  .at[128:128 + NHID, 3].set(a0_1[NHID:, 0]))
    wh1, sc1, mx1 = _project(x, w1, b1, amat1)
    er1_t = jnp.zeros((8, N), f32).at[0].set(sc1[:, 1]).at[1].set(sc1[:, 3])
    x1 = _attention(adj, wh1, sc1, mx1, er1_t, nheads=2, final=False)

    w2 = jnp.zeros((2 * NHID, 128), f32).at[:, 0:NHID].set(W_out)
    b2 = jnp.zeros((1, 128), f32).at[0, NHID].set(1.0)
    amat2 = jnp.zeros((128, 128), f32)
    amat2 = (amat2.at[0:NHID, 0].set(a_out[:NHID, 0])
                  .at[0:NHID, 1].set(a_out[NHID:, 0]))
    wh2, sc2, mx2 = _project(x1, w2, b2, amat2)
    er2_t = jnp.zeros((8, N), f32).at[0].set(sc2[:, 1])
    return _attention(adj, wh2, sc2, mx2, er2_t, nheads=1, final=True)


# speedup vs baseline: 2.0708x; 1.4357x over previous
import functools

import jax
import jax.numpy as jnp
from jax.experimental import pallas as pl
from jax.experimental.pallas import tpu as pltpu

N = 8192
NFEAT = 256
NHID = 64
ALPHA = 0.2


def _elu(x):
    return jnp.where(x > 0, x, jnp.exp(x) - 1.0)


def _proj_kernel(x_ref, w_ref, b_ref, amat_ref, h_ref, sc_ref, mx_ref):
    i = pl.program_id(0)
    h = jnp.dot(x_ref[...], w_ref[...], preferred_element_type=jnp.float32)
    h = h + b_ref[...]
    h_ref[...] = h.astype(jnp.bfloat16)
    sc = jnp.dot(h, amat_ref[...], preferred_element_type=jnp.float32)
    sc_ref[...] = sc
    cm = jnp.max(sc, axis=0, keepdims=True)

    @pl.when(i == 0)
    def _():
        mx_ref[...] = cm

    @pl.when(i > 0)
    def _():
        mx_ref[...] = jnp.maximum(mx_ref[...], cm)


def _project(x, w_aug, bias, amat, block_rows=1024):
    n, k = x.shape
    f = w_aug.shape[1]
    grid = (n // block_rows,)
    return pl.pallas_call(
        _proj_kernel,
        grid=grid,
        in_specs=[
            pl.BlockSpec((block_rows, k), lambda i: (i, 0)),
            pl.BlockSpec((k, f), lambda i: (0, 0)),
            pl.BlockSpec((1, f), lambda i: (0, 0)),
            pl.BlockSpec((f, 128), lambda i: (0, 0)),
        ],
        out_specs=[
            pl.BlockSpec((block_rows, f), lambda i: (i, 0)),
            pl.BlockSpec((block_rows, 128), lambda i: (i, 0)),
            pl.BlockSpec((1, 128), lambda i: (0, 0)),
        ],
        out_shape=[
            jax.ShapeDtypeStruct((n, f), jnp.bfloat16),
            jax.ShapeDtypeStruct((n, 128), jnp.float32),
            jax.ShapeDtypeStruct((1, 128), jnp.float32),
        ],
        compiler_params=pltpu.CompilerParams(
            dimension_semantics=("arbitrary",),
        ),
    )(x, w_aug, bias, amat)


def _attn_kernel(adj_ref, wh_ref, sc_ref, mx_ref, ert_ref, out_ref, acc_ref,
                 *, nheads, nj, bc, final):
    j = pl.program_id(1)

    @pl.when(j == 0)
    def _():
        acc_ref[...] = jnp.zeros_like(acc_ref)

    adj = adj_ref[...].astype(jnp.bfloat16)
    for h in range(nheads):
        el = sc_ref[:, 2 * h:2 * h + 1]
        erm = mx_ref[0:1, 2 * h + 1:2 * h + 2]
        t = el + erm
        m = jnp.maximum(t, ALPHA * t)
        a = jnp.exp(t - m).astype(jnp.bfloat16)
        c = jnp.exp(ALPHA * t - m).astype(jnp.bfloat16)
        d0 = ert_ref[h:h + 1, :] - erm
        b = jnp.exp(d0).astype(jnp.bfloat16)
        d = jnp.exp(ALPHA * d0).astype(jnp.bfloat16)
        p = adj * jnp.maximum(a * b, c * d)
        wh = wh_ref[pl.ds(j * bc, bc), 128 * h:128 * (h + 1)]
        acc_ref[:, 128 * h:128 * (h + 1)] += jnp.dot(
            p, wh, preferred_element_type=jnp.float32)

    @pl.when(j == nj - 1)
    def _():
        for h in range(nheads):
            acc = acc_ref[:, 128 * h:128 * (h + 1)]
            s = jnp.maximum(acc[:, NHID:NHID + 1], 1e-30)
            hp = acc[:, :NHID] * (1.0 / s)
            if final:
                o = _elu(hp)
                mx = jnp.max(o, axis=1, keepdims=True)
                lse = jnp.log(jnp.sum(jnp.exp(o - mx), axis=1, keepdims=True))
                out_ref[...] = o - mx - lse
            else:
                out_ref[:, NHID * h:NHID * (h + 1)] = _elu(hp)


def _attention(adj, wh_aug, scores, colmax, er_t, nheads, final,
               br=256, bc=2048):
    n = adj.shape[0]
    ni, nj = n // br, n // bc
    out_cols = NHID if final else NHID * nheads
    kern = functools.partial(_attn_kernel, nheads=nheads, nj=nj, bc=bc,
                             final=final)
    return pl.pallas_call(
        kern,
        grid=(ni, nj),
        in_specs=[
            pl.BlockSpec((br, bc), lambda i, j: (i, j)),
            pl.BlockSpec((n, 128 * nheads), lambda i, j: (0, 0)),
            pl.BlockSpec((br, 128), lambda i, j: (i, 0)),
            pl.BlockSpec((1, 128), lambda i, j: (0, 0)),
            pl.BlockSpec((8, bc), lambda i, j: (0, j)),
        ],
        out_specs=pl.BlockSpec((br, out_cols), lambda i, j: (i, 0)),
        out_shape=jax.ShapeDtypeStruct((n, out_cols), jnp.float32),
        scratch_shapes=[pltpu.VMEM((br, 128 * nheads), jnp.float32)],
        compiler_params=pltpu.CompilerParams(
            dimension_semantics=("parallel", "arbitrary"),
        ),
    )(adj, wh_aug, scores, colmax, er_t)


def kernel(x, adj, W0_0, a0_0, W0_1, a0_1, W_out, a_out):
    f32 = jnp.float32

    w1 = jnp.zeros((NFEAT, 256), f32)
    w1 = w1.at[:, 0:NHID].set(W0_0).at[:, 128:128 + NHID].set(W0_1)
    b1 = jnp.zeros((1, 256), f32).at[0, NHID].set(1.0).at[0, 128 + NHID].set(1.0)
    amat1 = jnp.zeros((256, 128), f32)
    amat1 = (amat1.at[0:NHID, 0].set(a0_0[:NHID, 0])
                  .at[0:NHID, 1].set(a0_0[NHID:, 0])
                  .at[128:128 + NHID, 2].set(a0_1[:NHID, 0])
                  .at[128:128 + NHID, 3].set(a0_1[NHID:, 0]))
    wh1, sc1, mx1 = _project(x, w1, b1, amat1)
    er1_t = jnp.zeros((8, N), f32).at[0].set(sc1[:, 1]).at[1].set(sc1[:, 3])
    x1 = _attention(adj, wh1, sc1, mx1, er1_t, nheads=2, final=False)

    w2 = jnp.zeros((2 * NHID, 128), f32).at[:, 0:NHID].set(W_out)
    b2 = jnp.zeros((1, 128), f32).at[0, NHID].set(1.0)
    amat2 = jnp.zeros((128, 128), f32)
    amat2 = (amat2.at[0:NHID, 0].set(a_out[:NHID, 0])
                  .at[0:NHID, 1].set(a_out[NHID:, 0]))
    wh2, sc2, mx2 = _project(x1, w2, b2, amat2)
    er2_t = jnp.zeros((8, N), f32).at[0].set(sc2[:, 1])
    return _attention(adj, wh2, sc2, mx2, er2_t, nheads=1, final=True)


# speedup vs baseline: 2.1530x; 1.0397x over previous
import functools

import jax
import jax.numpy as jnp
from jax.experimental import pallas as pl
from jax.experimental.pallas import tpu as pltpu

N = 8192
NFEAT = 256
NHID = 64
ALPHA = 0.2


def _elu(x):
    return jnp.where(x > 0, x, jnp.exp(x) - 1.0)


def _proj_kernel(x_ref, w_ref, b_ref, amat_ref, h_ref, sc_ref, mx_ref):
    i = pl.program_id(0)
    h = jnp.dot(x_ref[...], w_ref[...], preferred_element_type=jnp.float32)
    h = h + b_ref[...]
    h_ref[...] = h.astype(jnp.bfloat16)
    sc = jnp.dot(h, amat_ref[...], preferred_element_type=jnp.float32)
    sc_ref[...] = sc
    cm = jnp.max(sc, axis=0, keepdims=True)

    @pl.when(i == 0)
    def _():
        mx_ref[...] = cm

    @pl.when(i > 0)
    def _():
        mx_ref[...] = jnp.maximum(mx_ref[...], cm)


def _project(x, w_aug, bias, amat, block_rows=1024):
    n, k = x.shape
    f = w_aug.shape[1]
    grid = (n // block_rows,)
    return pl.pallas_call(
        _proj_kernel,
        grid=grid,
        in_specs=[
            pl.BlockSpec((block_rows, k), lambda i: (i, 0)),
            pl.BlockSpec((k, f), lambda i: (0, 0)),
            pl.BlockSpec((1, f), lambda i: (0, 0)),
            pl.BlockSpec((f, 128), lambda i: (0, 0)),
        ],
        out_specs=[
            pl.BlockSpec((block_rows, f), lambda i: (i, 0)),
            pl.BlockSpec((block_rows, 128), lambda i: (i, 0)),
            pl.BlockSpec((1, 128), lambda i: (0, 0)),
        ],
        out_shape=[
            jax.ShapeDtypeStruct((n, f), jnp.bfloat16),
            jax.ShapeDtypeStruct((n, 128), jnp.float32),
            jax.ShapeDtypeStruct((1, 128), jnp.float32),
        ],
        compiler_params=pltpu.CompilerParams(
            dimension_semantics=("arbitrary",),
        ),
    )(x, w_aug, bias, amat)


def _attn_kernel(adj_ref, wh_ref, sc_ref, mx_ref, ert_ref, *rest,
                 nheads, nj, bc, final, emit_i8):
    if emit_i8:
        out_ref, adj8_ref, acc_ref = rest
    else:
        out_ref, acc_ref = rest
    j = pl.program_id(1)

    @pl.when(j == 0)
    def _():
        acc_ref[...] = jnp.zeros_like(acc_ref)

    adj = adj_ref[...].astype(jnp.bfloat16)
    if emit_i8:
        adj8_ref[...] = adj_ref[...].astype(jnp.int8)
    for h in range(nheads):
        erm = mx_ref[0:1, 2 * h + 1:2 * h + 2]
        t = sc_ref[:, 2 * h:2 * h + 1] + erm
        g = jnp.minimum(jnp.exp(-0.8 * t), 3e17).astype(jnp.bfloat16)
        d0 = ert_ref[h:h + 1, :] - erm
        b = jnp.exp(d0).astype(jnp.bfloat16)
        d = jnp.exp(ALPHA * d0).astype(jnp.bfloat16)
        p = adj * jnp.maximum(b, g * d)
        wh = wh_ref[pl.ds(j * bc, bc), 128 * h:128 * (h + 1)]
        acc_ref[:, 128 * h:128 * (h + 1)] += jnp.dot(
            p, wh, preferred_element_type=jnp.float32)

    @pl.when(j == nj - 1)
    def _():
        for h in range(nheads):
            acc = acc_ref[:, 128 * h:128 * (h + 1)]
            s = jnp.maximum(acc[:, NHID:NHID + 1], 1e-30)
            hp = acc[:, :NHID] * (1.0 / s)
            if final:
                o = _elu(hp)
                mx = jnp.max(o, axis=1, keepdims=True)
                lse = jnp.log(jnp.sum(jnp.exp(o - mx), axis=1, keepdims=True))
                out_ref[...] = o - mx - lse
            else:
                out_ref[:, NHID * h:NHID * (h + 1)] = _elu(hp)


def _attention(adj, wh_aug, scores, colmax, er_t, nheads, final, emit_i8,
               br=256, bc=2048):
    n = adj.shape[0]
    ni, nj = n // br, n // bc
    out_cols = NHID if final else NHID * nheads
    kern = functools.partial(_attn_kernel, nheads=nheads, nj=nj, bc=bc,
                             final=final, emit_i8=emit_i8)
    out_specs = [pl.BlockSpec((br, out_cols), lambda i, j: (i, 0))]
    out_shape = [jax.ShapeDtypeStruct((n, out_cols), jnp.float32)]
    if emit_i8:
        out_specs.append(pl.BlockSpec((br, bc), lambda i, j: (i, j)))
        out_shape.append(jax.ShapeDtypeStruct((n, n), jnp.int8))
    return pl.pallas_call(
        kern,
        grid=(ni, nj),
        in_specs=[
            pl.BlockSpec((br, bc), lambda i, j: (i, j)),
            pl.BlockSpec((n, 128 * nheads), lambda i, j: (0, 0)),
            pl.BlockSpec((br, 128), lambda i, j: (i, 0)),
            pl.BlockSpec((1, 128), lambda i, j: (0, 0)),
            pl.BlockSpec((8, bc), lambda i, j: (0, j)),
        ],
        out_specs=out_specs,
        out_shape=out_shape,
        scratch_shapes=[pltpu.VMEM((br, 128 * nheads), jnp.float32)],
        compiler_params=pltpu.CompilerParams(
            dimension_semantics=("parallel", "arbitrary"),
        ),
    )(adj, wh_aug, scores, colmax, er_t)


def kernel(x, adj, W0_0, a0_0, W0_1, a0_1, W_out, a_out):
    f32 = jnp.float32

    w1 = jnp.zeros((NFEAT, 256), f32)
    w1 = w1.at[:, 0:NHID].set(W0_0).at[:, 128:128 + NHID].set(W0_1)
    b1 = jnp.zeros((1, 256), f32).at[0, NHID].set(1.0).at[0, 128 + NHID].set(1.0)
    amat1 = jnp.zeros((256, 128), f32)
    amat1 = (amat1.at[0:NHID, 0].set(a0_0[:NHID, 0])
                  .at[0:NHID, 1].set(a0_0[NHID:, 0])
                  .at[128:128 + NHID, 2].set(a0_1[:NHID, 0])
                  .at[128:128 + NHID, 3].set(a0_1[NHID:, 0]))
    wh1, sc1, mx1 = _project(x, w1, b1, amat1)
    er1_t = jnp.zeros((8, N), f32).at[0].set(sc1[:, 1]).at[1].set(sc1[:, 3])
    x1, adj8 = _attention(adj, wh1, sc1, mx1, er1_t, nheads=2, final=False,
                          emit_i8=True)

    w2 = jnp.zeros((2 * NHID, 128), f32).at[:, 0:NHID].set(W_out)
    b2 = jnp.zeros((1, 128), f32).at[0, NHID].set(1.0)
    amat2 = jnp.zeros((128, 128), f32)
    amat2 = (amat2.at[0:NHID, 0].set(a_out[:NHID, 0])
                  .at[0:NHID, 1].set(a_out[NHID:, 0]))
    wh2, sc2, mx2 = _project(x1, w2, b2, amat2)
    er2_t = jnp.zeros((8, N), f32).at[0].set(sc2[:, 1])
    (out,) = _attention(adj8, wh2, sc2, mx2, er2_t, nheads=1, final=True,
                        emit_i8=False)
    return out


# speedup vs baseline: 2.8594x; 1.3281x over previous
import functools

import jax
import jax.numpy as jnp
from jax.experimental import pallas as pl
from jax.experimental.pallas import tpu as pltpu

N = 8192
NFEAT = 256
NHID = 64
ALPHA = 0.2


def _elu(x):
    return jnp.where(x > 0, x, jnp.exp(x) - 1.0)


def _proj_kernel(x_ref, w_ref, b_ref, amat_ref, h_ref, sc_ref, mx_ref):
    i = pl.program_id(0)
    h = jnp.dot(x_ref[...], w_ref[...], preferred_element_type=jnp.float32)
    h = h + b_ref[...]
    h_ref[...] = h.astype(jnp.bfloat16)
    sc = jnp.dot(h, amat_ref[...], preferred_element_type=jnp.float32)
    sc_ref[...] = sc
    cm = jnp.max(sc, axis=0, keepdims=True)

    @pl.when(i == 0)
    def _():
        mx_ref[...] = cm

    @pl.when(i > 0)
    def _():
        mx_ref[...] = jnp.maximum(mx_ref[...], cm)


def _project(x, w_aug, bias, amat, block_rows=1024):
    n, k = x.shape
    f = w_aug.shape[1]
    grid = (n // block_rows,)
    return pl.pallas_call(
        _proj_kernel,
        grid=grid,
        in_specs=[
            pl.BlockSpec((block_rows, k), lambda i: (i, 0)),
            pl.BlockSpec((k, f), lambda i: (0, 0)),
            pl.BlockSpec((1, f), lambda i: (0, 0)),
            pl.BlockSpec((f, 128), lambda i: (0, 0)),
        ],
        out_specs=[
            pl.BlockSpec((block_rows, f), lambda i: (i, 0)),
            pl.BlockSpec((block_rows, 128), lambda i: (i, 0)),
            pl.BlockSpec((1, 128), lambda i: (0, 0)),
        ],
        out_shape=[
            jax.ShapeDtypeStruct((n, f), jnp.bfloat16),
            jax.ShapeDtypeStruct((n, 128), jnp.float32),
            jax.ShapeDtypeStruct((1, 128), jnp.float32),
        ],
        compiler_params=pltpu.CompilerParams(
            dimension_semantics=("arbitrary",),
        ),
    )(x, w_aug, bias, amat)


def _attn_kernel(adj_ref, wh_ref, sc_ref, mx_ref, ert_ref, *rest,
                 nheads, nj, bc, final, emit_i8):
    if emit_i8:
        out_ref, adj8_ref, acc_ref = rest
    else:
        out_ref, acc_ref = rest
    j = pl.program_id(1)

    @pl.when(j == 0)
    def _():
        acc_ref[...] = jnp.zeros_like(acc_ref)

    adj = adj_ref[...].astype(jnp.bfloat16)
    if emit_i8:
        adj8_ref[...] = adj.astype(jnp.int8)
    for h in range(nheads):
        erm = mx_ref[0:1, 2 * h + 1:2 * h + 2]
        t = sc_ref[:, 2 * h:2 * h + 1] + erm
        g = jnp.minimum(jnp.exp(-0.8 * t), 3e17).astype(jnp.bfloat16)
        d0 = ert_ref[h:h + 1, :] - erm
        b = jnp.exp(d0).astype(jnp.bfloat16)
        d = jnp.exp(ALPHA * d0).astype(jnp.bfloat16)
        p = adj * jnp.maximum(b, g * d)
        wh = wh_ref[pl.ds(j * bc, bc), 128 * h:128 * (h + 1)]
        acc_ref[:, 128 * h:128 * (h + 1)] += jnp.dot(
            p, wh, preferred_element_type=jnp.float32)

    @pl.when(j == nj - 1)
    def _():
        for h in range(nheads):
            acc = acc_ref[:, 128 * h:128 * (h + 1)]
            s = jnp.maximum(acc[:, NHID:NHID + 1], 1e-30)
            hp = acc[:, :NHID] * (1.0 / s)
            if final:
                o = _elu(hp)
                mx = jnp.max(o, axis=1, keepdims=True)
                lse = jnp.log(jnp.sum(jnp.exp(o - mx), axis=1, keepdims=True))
                out_ref[...] = o - mx - lse
            else:
                out_ref[:, NHID * h:NHID * (h + 1)] = _elu(hp)


def _attention(adj, wh_aug, scores, colmax, er_t, nheads, final, emit_i8,
               br=512, bc=2048):
    n = adj.shape[0]
    ni, nj = n // br, n // bc
    out_cols = NHID if final else NHID * nheads
    kern = functools.partial(_attn_kernel, nheads=nheads, nj=nj, bc=bc,
                             final=final, emit_i8=emit_i8)
    out_specs = [pl.BlockSpec((br, out_cols), lambda i, j: (i, 0))]
    out_shape = [jax.ShapeDtypeStruct((n, out_cols), jnp.float32)]
    if emit_i8:
        out_specs.append(pl.BlockSpec((br, bc), lambda i, j: (i, j)))
        out_shape.append(jax.ShapeDtypeStruct((n, n), jnp.int8))
    return pl.pallas_call(
        kern,
        grid=(ni, nj),
        in_specs=[
            pl.BlockSpec((br, bc), lambda i, j: (i, j)),
            pl.BlockSpec((n, 128 * nheads), lambda i, j: (0, 0)),
            pl.BlockSpec((br, 128), lambda i, j: (i, 0)),
            pl.BlockSpec((1, 128), lambda i, j: (0, 0)),
            pl.BlockSpec((8, bc), lambda i, j: (0, j)),
        ],
        out_specs=out_specs,
        out_shape=out_shape,
        scratch_shapes=[pltpu.VMEM((br, 128 * nheads), jnp.float32)],
        compiler_params=pltpu.CompilerParams(
            dimension_semantics=("parallel", "arbitrary"),
        ),
    )(adj, wh_aug, scores, colmax, er_t)


def kernel(x, adj, W0_0, a0_0, W0_1, a0_1, W_out, a_out):
    f32 = jnp.float32

    w1 = jnp.zeros((NFEAT, 256), f32)
    w1 = w1.at[:, 0:NHID].set(W0_0).at[:, 128:128 + NHID].set(W0_1)
    b1 = jnp.zeros((1, 256), f32).at[0, NHID].set(1.0).at[0, 128 + NHID].set(1.0)
    amat1 = jnp.zeros((256, 128), f32)
    amat1 = (amat1.at[0:NHID, 0].set(a0_0[:NHID, 0])
                  .at[0:NHID, 1].set(a0_0[NHID:, 0])
                  .at[128:128 + NHID, 2].set(a0_1[:NHID, 0])
                  .at[128:128 + NHID, 3].set(a0_1[NHID:, 0]))
    wh1, sc1, mx1 = _project(x, w1, b1, amat1)
    er1_t = jnp.zeros((8, N), f32).at[0].set(sc1[:, 1]).at[1].set(sc1[:, 3])
    x1, adj8 = _attention(adj, wh1, sc1, mx1, er1_t, nheads=2, final=False,
                          emit_i8=True)

    w2 = jnp.zeros((2 * NHID, 128), f32).at[:, 0:NHID].set(W_out)
    b2 = jnp.zeros((1, 128), f32).at[0, NHID].set(1.0)
    amat2 = jnp.zeros((128, 128), f32)
    amat2 = (amat2.at[0:NHID, 0].set(a_out[:NHID, 0])
                  .at[0:NHID, 1].set(a_out[NHID:, 0]))
    wh2, sc2, mx2 = _project(x1, w2, b2, amat2)
    er2_t = jnp.zeros((8, N), f32).at[0].set(sc2[:, 1])
    (out,) = _attention(adj8, wh2, sc2, mx2, er2_t, nheads=1, final=True,
                        emit_i8=False)
    return out


# speedup vs baseline: 3.0743x; 1.0752x over previous
import functools

import jax
import jax.numpy as jnp
from jax.experimental import pallas as pl
from jax.experimental.pallas import tpu as pltpu

N = 8192
NFEAT = 256
NHID = 64
ALPHA = 0.2


def _elu(x):
    return jnp.where(x > 0, x, jnp.exp(x) - 1.0)


def _proj_kernel(x_ref, w_ref, b_ref, amat_ref, h_ref, sc_ref, mx_ref):
    i = pl.program_id(0)
    h = jnp.dot(x_ref[...], w_ref[...], preferred_element_type=jnp.float32)
    h = h + b_ref[...]
    h_ref[...] = h.astype(jnp.float8_e4m3fn)
    sc = jnp.dot(h, amat_ref[...], preferred_element_type=jnp.float32)
    sc_ref[...] = sc
    cm = jnp.max(sc, axis=0, keepdims=True)

    @pl.when(i == 0)
    def _():
        mx_ref[...] = cm

    @pl.when(i > 0)
    def _():
        mx_ref[...] = jnp.maximum(mx_ref[...], cm)


def _project(x, w_aug, bias, amat, block_rows=1024):
    n, k = x.shape
    f = w_aug.shape[1]
    grid = (n // block_rows,)
    return pl.pallas_call(
        _proj_kernel,
        grid=grid,
        in_specs=[
            pl.BlockSpec((block_rows, k), lambda i: (i, 0)),
            pl.BlockSpec((k, f), lambda i: (0, 0)),
            pl.BlockSpec((1, f), lambda i: (0, 0)),
            pl.BlockSpec((f, 128), lambda i: (0, 0)),
        ],
        out_specs=[
            pl.BlockSpec((block_rows, f), lambda i: (i, 0)),
            pl.BlockSpec((block_rows, 128), lambda i: (i, 0)),
            pl.BlockSpec((1, 128), lambda i: (0, 0)),
        ],
        out_shape=[
            jax.ShapeDtypeStruct((n, f), jnp.float8_e4m3fn),
            jax.ShapeDtypeStruct((n, 128), jnp.float32),
            jax.ShapeDtypeStruct((1, 128), jnp.float32),
        ],
        compiler_params=pltpu.CompilerParams(
            dimension_semantics=("arbitrary",),
        ),
    )(x, w_aug, bias, amat)


def _attn_kernel(adj_ref, wh_ref, sc_ref, mx_ref, ert_ref, *rest,
                 nheads, nj, bc, final, emit_i8):
    if emit_i8:
        out_ref, adj8_ref, acc_ref = rest
    else:
        out_ref, acc_ref = rest
    j = pl.program_id(1)

    @pl.when(j == 0)
    def _():
        acc_ref[...] = jnp.zeros_like(acc_ref)

    adj = adj_ref[...].astype(jnp.bfloat16)
    if emit_i8:
        adj8_ref[...] = adj.astype(jnp.int8)
    for h in range(nheads):
        erm = mx_ref[0:1, 2 * h + 1:2 * h + 2]
        t = sc_ref[:, 2 * h:2 * h + 1] + erm
        g = jnp.minimum(jnp.exp(-0.8 * t), 240.0).astype(jnp.bfloat16)
        d0 = ert_ref[h:h + 1, :] - erm
        b = jnp.exp(d0).astype(jnp.bfloat16)
        d = jnp.exp(ALPHA * d0).astype(jnp.bfloat16)
        p = adj * jnp.maximum(b, g * d)
        wh = wh_ref[pl.ds(j * bc, bc), 128 * h:128 * (h + 1)]
        acc_ref[:, 128 * h:128 * (h + 1)] += jnp.dot(
            p.astype(jnp.float8_e4m3fn), wh,
            preferred_element_type=jnp.float32)

    @pl.when(j == nj - 1)
    def _():
        for h in range(nheads):
            acc = acc_ref[:, 128 * h:128 * (h + 1)]
            s = jnp.maximum(acc[:, NHID:NHID + 1], 1e-30)
            hp = acc[:, :NHID] * (1.0 / s)
            if final:
                o = _elu(hp)
                mx = jnp.max(o, axis=1, keepdims=True)
                lse = jnp.log(jnp.sum(jnp.exp(o - mx), axis=1, keepdims=True))
                out_ref[...] = o - mx - lse
            else:
                out_ref[:, NHID * h:NHID * (h + 1)] = _elu(hp)


def _attention(adj, wh_aug, scores, colmax, er_t, nheads, final, emit_i8,
               br=512, bc=2048):
    n = adj.shape[0]
    ni, nj = n // br, n // bc
    out_cols = NHID if final else NHID * nheads
    kern = functools.partial(_attn_kernel, nheads=nheads, nj=nj, bc=bc,
                             final=final, emit_i8=emit_i8)
    out_specs = [pl.BlockSpec((br, out_cols), lambda i, j: (i, 0))]
    out_shape = [jax.ShapeDtypeStruct((n, out_cols), jnp.float32)]
    if emit_i8:
        out_specs.append(pl.BlockSpec((br, bc), lambda i, j: (i, j)))
        out_shape.append(jax.ShapeDtypeStruct((n, n), jnp.int8))
    return pl.pallas_call(
        kern,
        grid=(ni, nj),
        in_specs=[
            pl.BlockSpec((br, bc), lambda i, j: (i, j)),
            pl.BlockSpec((n, 128 * nheads), lambda i, j: (0, 0)),
            pl.BlockSpec((br, 128), lambda i, j: (i, 0)),
            pl.BlockSpec((1, 128), lambda i, j: (0, 0)),
            pl.BlockSpec((8, bc), lambda i, j: (0, j)),
        ],
        out_specs=out_specs,
        out_shape=out_shape,
        scratch_shapes=[pltpu.VMEM((br, 128 * nheads), jnp.float32)],
        compiler_params=pltpu.CompilerParams(
            dimension_semantics=("parallel", "arbitrary"),
        ),
    )(adj, wh_aug, scores, colmax, er_t)


def kernel(x, adj, W0_0, a0_0, W0_1, a0_1, W_out, a_out):
    f32 = jnp.float32

    w1 = jnp.zeros((NFEAT, 256), f32)
    w1 = w1.at[:, 0:NHID].set(W0_0).at[:, 128:128 + NHID].set(W0_1)
    b1 = jnp.zeros((1, 256), f32).at[0, NHID].set(1.0).at[0, 128 + NHID].set(1.0)
    amat1 = jnp.zeros((256, 128), f32)
    amat1 = (amat1.at[0:NHID, 0].set(a0_0[:NHID, 0])
                  .at[0:NHID, 1].set(a0_0[NHID:, 0])
                  .at[128:128 + NHID, 2].set(a0_1[:NHID, 0])
                  .at[128:128 + NHID, 3].set(a0_1[NHID:, 0]))
    wh1, sc1, mx1 = _project(x, w1, b1, amat1)
    er1_t = jnp.zeros((8, N), f32).at[0].set(sc1[:, 1]).at[1].set(sc1[:, 3])
    x1, adj8 = _attention(adj, wh1, sc1, mx1, er1_t, nheads=2, final=False,
                          emit_i8=True)

    w2 = jnp.zeros((2 * NHID, 128), f32).at[:, 0:NHID].set(W_out)
    b2 = jnp.zeros((1, 128), f32).at[0, NHID].set(1.0)
    amat2 = jnp.zeros((128, 128), f32)
    amat2 = (amat2.at[0:NHID, 0].set(a_out[:NHID, 0])
                  .at[0:NHID, 1].set(a_out[NHID:, 0]))
    wh2, sc2, mx2 = _project(x1, w2, b2, amat2)
    er2_t = jnp.zeros((8, N), f32).at[0].set(sc2[:, 1])
    (out,) = _attention(adj8, wh2, sc2, mx2, er2_t, nheads=1, final=True,
                        emit_i8=False)
    return out
